# baseline (speedup 1.0000x reference)
import functools
import jax
import jax.numpy as jnp
from jax import lax
from jax.experimental import pallas as pl
from jax.experimental.pallas import tpu as pltpu
from jax.experimental.pallas import tpu_sc as plsc

N = 10000
E = 320000
IN = 128
SEM_HID = 128
OUT = 64

NTILES = 16
ROWS_PER_TILE = 640
NPAD = NTILES * ROWS_PER_TILE
CHUNK = 80
NBUF = 4
GSTAGE = 3
STAGE = GSTAGE * NBUF
STAGES = 21
CHUNKS = STAGE * STAGES
GROUPS = CHUNKS // NBUF
EPT = CHUNKS * CHUNK
EPAD = NTILES * EPT

f32 = jnp.float32
i32 = jnp.int32

_MESH = plsc.VectorSubcoreMesh(core_axis_name="c", subcore_axis_name="s")


@functools.partial(
    pl.kernel,
    out_type=[jax.ShapeDtypeStruct((NPAD,), f32)] * 4,
    mesh=_MESH,
    scratch_types=[
        pltpu.VMEM((STAGE, CHUNK), i32),
        pltpu.VMEM((112,), f32),
        pltpu.VMEM_SHARED((NPAD,), f32),
        pltpu.VMEM_SHARED((NPAD,), f32),
        pltpu.SemaphoreType.DMA,
    ],
)
def _hist_kernel(src0, dst0, src1, dst1, z1, ds0, di0, ds1, di1,
                 idx_v, ones_v, hsrc, hdst, hsem):
    c = lax.axis_index("c")
    s = lax.axis_index("s")
    for i in range(112 // 16):
        ones_v[pl.ds(i * 16, 16)] = jnp.ones((16,), f32)
    ones_c = ones_v.at[pl.ds(0, CHUNK)]
    sl = pl.ds(s * ROWS_PER_TILE, ROWS_PER_TILE)
    pltpu.sync_copy(z1, hsrc.at[sl])
    pltpu.sync_copy(z1, hdst.at[sl])
    plsc.subcore_barrier()

    def accumulate(edges_hbm, hist, sem):
        def stage_body(st, carry):
            pltpu.sync_copy(edges_hbm.at[s * STAGES + st], idx_v)

            def body(j, carry2):
                pltpu.async_copy(ones_c, hist.at[idx_v.at[j]], sem, add=True)
                return carry2
            lax.fori_loop(0, STAGE, body, 0)

            def drain(j, carry2):
                pltpu.make_async_copy(ones_c, hist.at[idx_v.at[0]], sem).wait()
                return carry2
            lax.fori_loop(0, STAGE, drain, 0)
            return carry
        lax.fori_loop(0, STAGES, stage_body, 0)

    @pl.when(c == 0)
    def _():
        accumulate(src0, hsrc, hsem)
        accumulate(dst0, hdst, hsem)

    @pl.when(c == 1)
    def _():
        accumulate(src1, hsrc, hsem)
        accumulate(dst1, hdst, hsem)

    plsc.subcore_barrier()

    @pl.when(c == 0)
    def _():
        pltpu.sync_copy(hsrc.at[sl], ds0.at[sl])
        pltpu.sync_copy(hdst.at[sl], di0.at[sl])

    @pl.when(c == 1)
    def _():
        pltpu.sync_copy(hsrc.at[sl], ds1.at[sl])
        pltpu.sync_copy(hdst.at[sl], di1.at[sl])


@functools.partial(
    pl.kernel,
    out_type=[jax.ShapeDtypeStruct((NPAD, IN), f32)] * 2,
    mesh=_MESH,
    scratch_types=[
        pltpu.VMEM((STAGE, CHUNK), i32),
        pltpu.VMEM((STAGE, CHUNK), i32),
        [pltpu.VMEM((CHUNK, IN), f32)] * NBUF,
        [pltpu.SemaphoreType.DMA] * NBUF,
        [pltpu.SemaphoreType.DMA] * NBUF,
        pltpu.VMEM_SHARED((NPAD, IN), f32),
    ],
)
def _scatter_kernel(xs0, xs1, src0, dst0, src1, dst1, z2, y0, y1,
                    src_v, dst_v, bufs, gsems, ssems, ys):
    c = lax.axis_index("c")
    s = lax.axis_index("s")
    sl = pl.ds(s * ROWS_PER_TILE, ROWS_PER_TILE)
    pltpu.sync_copy(z2, ys.at[sl])
    plsc.subcore_barrier()

    def run(xs_hbm, src_hbm, dst_hbm):
        def group(gi, carry):
            at_stage = gi % GSTAGE == 0

            @pl.when(at_stage)
            def _():
                @pl.when(gi > 0)
                def _():
                    for b in range(NBUF):
                        pltpu.make_async_copy(
                            bufs[b], ys.at[dst_v.at[b]], ssems[b]).wait()
                st = gi // GSTAGE
                pltpu.sync_copy(src_hbm.at[s * STAGES + st], src_v)
                pltpu.sync_copy(dst_hbm.at[s * STAGES + st], dst_v)

            jj = (gi % GSTAGE) * NBUF
            for b in range(NBUF):
                @pl.when(jnp.logical_not(at_stage))
                def _():
                    pltpu.make_async_copy(
                        bufs[b], ys.at[dst_v.at[jj + b]], ssems[b]).wait()
                pltpu.async_copy(xs_hbm.at[src_v.at[jj + b]], bufs[b], gsems[b])
            for b in range(NBUF):
                pltpu.make_async_copy(
                    xs_hbm.at[src_v.at[jj + b]], bufs[b], gsems[b]).wait()
                pltpu.async_copy(bufs[b], ys.at[dst_v.at[jj + b]], ssems[b],
                                 add=True)
            return carry
        lax.fori_loop(0, GROUPS, group, 0)
        for b in range(NBUF):
            pltpu.make_async_copy(
                bufs[b], ys.at[dst_v.at[b]], ssems[b]).wait()

    @pl.when(c == 0)
    def _():
        run(xs0, src0, dst0)

    @pl.when(c == 1)
    def _():
        run(xs1, src1, dst1)

    plsc.subcore_barrier()

    @pl.when(c == 0)
    def _():
        pltpu.sync_copy(ys.at[sl], y0.at[sl])

    @pl.when(c == 1)
    def _():
        pltpu.sync_copy(ys.at[sl], y1.at[sl])


def _scale_body(x_ref, d0_ref, d1_ref, xs0_ref, xs1_ref):
    x = x_ref[...]
    n0 = lax.rsqrt(jnp.maximum(d0_ref[...], 1.0))
    n1 = lax.rsqrt(jnp.maximum(d1_ref[...], 1.0))
    xs0_ref[...] = x * n0
    xs1_ref[...] = x * n1


_RB = 2560
_NB = NPAD // _RB


def _dense_body(y0_ref, y1_ref, d0_ref, d1_ref, Wg0_ref, Wg1_ref, Ws1_ref,
                bs1_ref, w2_ref, Wp_ref, bg0_ref, bg1_ref, bp_ref, o_ref,
                acc, M0s, M1s, k0s, k1s, G0s, G1s, g0s, g1s):
    p = pl.program_id(0)
    i = pl.program_id(1)

    @pl.when((p == 0) & (i == 0))
    def _():
        M0s[...] = jnp.dot(Wg0_ref[...], Ws1_ref[...],
                           preferred_element_type=f32)
        M1s[...] = jnp.dot(Wg1_ref[...], Ws1_ref[...],
                           preferred_element_type=f32)
        k0s[...] = jnp.dot(bg0_ref[...], Ws1_ref[...],
                           preferred_element_type=f32) + bs1_ref[...]
        k1s[...] = jnp.dot(bg1_ref[...], Ws1_ref[...],
                           preferred_element_type=f32) + bs1_ref[...]
        G0s[...] = jnp.dot(Wg0_ref[...], Wp_ref[...],
                           preferred_element_type=f32)
        G1s[...] = jnp.dot(Wg1_ref[...], Wp_ref[...],
                           preferred_element_type=f32)
        g0s[...] = jnp.dot(bg0_ref[...], Wp_ref[...],
                           preferred_element_type=f32)
        g1s[...] = jnp.dot(bg1_ref[...], Wp_ref[...],
                           preferred_element_type=f32)
        acc[...] = jnp.zeros((8, 128), f32)

    yb0 = y0_ref[...] * lax.rsqrt(jnp.maximum(d0_ref[...], 1.0))
    yb1 = y1_ref[...] * lax.rsqrt(jnp.maximum(d1_ref[...], 1.0))
    row = lax.broadcasted_iota(i32, (8, 128), 0)
    lane = lax.broadcasted_iota(i32, (8, 128), 1)

    @pl.when(p == 0)
    def _():
        rows = lax.broadcasted_iota(i32, (_RB, 1), 0) + i * _RB
        mask = rows < N

        def part(yb, M_ref, k_ref):
            a = jnp.tanh(jnp.dot(yb, M_ref[...], preferred_element_type=f32)
                         + k_ref[...])
            t = jnp.sum(a * w2_ref[...], axis=1, keepdims=True)
            return jnp.sum(jnp.where(mask, t, 0.0))

        s0 = part(yb0, M0s, k0s)
        s1 = part(yb1, M1s, k1s)
        acc[...] += (jnp.where((row == 0) & (lane == 0), s0, 0.0)
                     + jnp.where((row == 1) & (lane == 0), s1, 0.0))

    @pl.when(p == 1)
    def _():
        srow = acc[...]
        w0 = jnp.sum(jnp.where((row == 0) & (lane == 0), srow, 0.0)) / N
        w1 = jnp.sum(jnp.where((row == 1) & (lane == 0), srow, 0.0)) / N
        m = jnp.maximum(w0, w1)
        e0 = jnp.exp(w0 - m)
        e1 = jnp.exp(w1 - m)
        b0 = e0 / (e0 + e1)
        b1 = e1 / (e0 + e1)
        p0 = jnp.dot(yb0, G0s[...], preferred_element_type=f32) + g0s[...]
        p1 = jnp.dot(yb1, G1s[...], preferred_element_type=f32) + g1s[...]
        o_ref[...] = b0 * p0 + b1 * p1 + bp_ref[...]


def _pad_edges(idx):
    pad = jnp.full((EPAD - E,), N, dtype=i32)
    return jnp.concatenate([idx, pad]).reshape(NTILES * STAGES, STAGE, CHUNK)


def kernel(x, edge_index0, edge_index1, Wg0, bg0, Wg1, bg1,
           Ws1, bs1, Ws2, Wp, bp):
    src0 = _pad_edges(edge_index0[0])
    dst0 = _pad_edges(edge_index0[1])
    src1 = _pad_edges(edge_index1[0])
    dst1 = _pad_edges(edge_index1[1])
    x_pad = jnp.pad(x, ((0, NPAD - N), (0, 0)))
    z1 = jnp.zeros((ROWS_PER_TILE,), f32)
    z2 = jnp.zeros((ROWS_PER_TILE, IN), f32)

    ds0, di0, ds1, di1 = _hist_kernel(src0, dst0, src1, dst1, z1)

    xs0, xs1 = pl.pallas_call(
        _scale_body,
        grid=(_NB,),
        in_specs=[
            pl.BlockSpec((_RB, IN), lambda i: (i, 0)),
            pl.BlockSpec((_RB, 1), lambda i: (i, 0)),
            pl.BlockSpec((_RB, 1), lambda i: (i, 0)),
        ],
        out_specs=[
            pl.BlockSpec((_RB, IN), lambda i: (i, 0)),
            pl.BlockSpec((_RB, IN), lambda i: (i, 0)),
        ],
        out_shape=[jax.ShapeDtypeStruct((NPAD, IN), f32)] * 2,
    )(x_pad, ds0.reshape(NPAD, 1), ds1.reshape(NPAD, 1))

    y0, y1 = _scatter_kernel(xs0, xs1, src0, dst0, src1, dst1, z2)

    di0c = di0.reshape(NPAD, 1)
    di1c = di1.reshape(NPAD, 1)

    DH = Wg0.shape[1]
    blk = lambda p, i: (i, 0)
    full = lambda p, i: (0, 0)
    outp = pl.pallas_call(
        _dense_body,
        grid=(2, _NB),
        in_specs=[
            pl.BlockSpec((_RB, IN), blk),
            pl.BlockSpec((_RB, IN), blk),
            pl.BlockSpec((_RB, 1), blk),
            pl.BlockSpec((_RB, 1), blk),
            pl.BlockSpec((IN, DH), full),
            pl.BlockSpec((IN, DH), full),
            pl.BlockSpec((DH, SEM_HID), full),
            pl.BlockSpec((1, SEM_HID), full),
            pl.BlockSpec((1, SEM_HID), full),
            pl.BlockSpec((DH, OUT), full),
            pl.BlockSpec((1, DH), full),
            pl.BlockSpec((1, DH), full),
            pl.BlockSpec((1, OUT), full),
        ],
        out_specs=pl.BlockSpec((_RB, OUT), blk),
        out_shape=jax.ShapeDtypeStruct((NPAD, OUT), f32),
        scratch_shapes=[
            pltpu.VMEM((8, 128), f32),
            pltpu.VMEM((IN, SEM_HID), f32),
            pltpu.VMEM((IN, SEM_HID), f32),
            pltpu.VMEM((1, SEM_HID), f32),
            pltpu.VMEM((1, SEM_HID), f32),
            pltpu.VMEM((IN, OUT), f32),
            pltpu.VMEM((IN, OUT), f32),
            pltpu.VMEM((1, OUT), f32),
            pltpu.VMEM((1, OUT), f32),
        ],
    )(y0, y1, di0c, di1c, Wg0, Wg1, Ws1,
      bs1.reshape(1, SEM_HID), Ws2.reshape(1, SEM_HID), Wp,
      bg0.reshape(1, DH), bg1.reshape(1, DH), bp.reshape(1, OUT))

    return outp[:N]


# speedup vs baseline: 1.0916x; 1.0916x over previous
import functools
import jax
import jax.numpy as jnp
from jax import lax
from jax.experimental import pallas as pl
from jax.experimental.pallas import tpu as pltpu
from jax.experimental.pallas import tpu_sc as plsc

N = 10000
E = 320000
IN = 128
SEM_HID = 128
OUT = 64

NTILES = 16
ROWS_PER_TILE = 640
NPAD = NTILES * ROWS_PER_TILE
CHUNK = 112
NBUF = 3
GSTAGE = 6
STAGE = GSTAGE * NBUF
STAGES = 10
CHUNKS = STAGE * STAGES
GROUPS = CHUNKS // NBUF
EPT = CHUNKS * CHUNK
EPAD = NTILES * EPT

f32 = jnp.float32
i32 = jnp.int32

_MESH = plsc.VectorSubcoreMesh(core_axis_name="c", subcore_axis_name="s")


@functools.partial(
    pl.kernel,
    out_type=[jax.ShapeDtypeStruct((NPAD,), f32)] * 4,
    mesh=_MESH,
    scratch_types=[
        [pltpu.VMEM((STAGE, CHUNK), i32)] * 2,
        [pltpu.VMEM((STAGE, CHUNK), i32)] * 2,
        pltpu.VMEM((112,), f32),
        pltpu.VMEM_SHARED((NPAD,), f32),
        pltpu.VMEM_SHARED((NPAD,), f32),
        [pltpu.SemaphoreType.DMA] * 2,
        [pltpu.SemaphoreType.DMA] * 2,
    ],
)
def _hist_kernel(src0, dst0, src1, dst1, z1, ds0, di0, ds1, di1,
                 sidx, didx, ones_v, hsrc, hdst, ssem, dsem):
    c = lax.axis_index("c")
    s = lax.axis_index("s")
    for i in range(112 // 16):
        ones_v[pl.ds(i * 16, 16)] = jnp.ones((16,), f32)
    ones_c = ones_v.at[pl.ds(0, CHUNK)]
    sl = pl.ds(s * ROWS_PER_TILE, ROWS_PER_TILE)
    pltpu.sync_copy(z1, hsrc.at[sl])
    pltpu.sync_copy(z1, hdst.at[sl])
    plsc.subcore_barrier()

    def accumulate(src_hbm, dst_hbm):
        def drain(sv, dv, p):
            def one(j, carry):
                pltpu.make_async_copy(ones_c, hsrc.at[sv.at[0]],
                                      ssem[p]).wait()
                pltpu.make_async_copy(ones_c, hdst.at[dv.at[0]],
                                      dsem[p]).wait()
                return carry
            lax.fori_loop(0, STAGE, one, 0)

        for st in range(STAGES):
            p = st % 2
            if st >= 2:
                drain(sidx[p], didx[p], p)
            pltpu.sync_copy(src_hbm.at[s * STAGES + st], sidx[p])
            pltpu.sync_copy(dst_hbm.at[s * STAGES + st], didx[p])

            def fire(j, carry, sv=sidx[p], dv=didx[p], p=p):
                pltpu.async_copy(ones_c, hsrc.at[sv.at[j]], ssem[p], add=True)
                pltpu.async_copy(ones_c, hdst.at[dv.at[j]], dsem[p], add=True)
                return carry
            lax.fori_loop(0, STAGE, fire, 0)
        for p in range(2):
            drain(sidx[p], didx[p], p)

    @pl.when(c == 0)
    def _():
        accumulate(src0, dst0)

    @pl.when(c == 1)
    def _():
        accumulate(src1, dst1)

    plsc.subcore_barrier()

    @pl.when(c == 0)
    def _():
        pltpu.sync_copy(hsrc.at[sl], ds0.at[sl])
        pltpu.sync_copy(hdst.at[sl], di0.at[sl])

    @pl.when(c == 1)
    def _():
        pltpu.sync_copy(hsrc.at[sl], ds1.at[sl])
        pltpu.sync_copy(hdst.at[sl], di1.at[sl])


@functools.partial(
    pl.kernel,
    out_type=[jax.ShapeDtypeStruct((NPAD, IN), f32)] * 2,
    mesh=_MESH,
    scratch_types=[
        pltpu.VMEM((STAGE, CHUNK), i32),
        pltpu.VMEM((STAGE, CHUNK), i32),
        [pltpu.VMEM((CHUNK, IN), f32)] * NBUF,
        [pltpu.SemaphoreType.DMA] * NBUF,
        [pltpu.SemaphoreType.DMA] * NBUF,
        pltpu.VMEM_SHARED((NPAD, IN), f32),
    ],
)
def _scatter_kernel(xs0, xs1, src0, dst0, src1, dst1, z2, y0, y1,
                    src_v, dst_v, bufs, gsems, ssems, ys):
    c = lax.axis_index("c")
    s = lax.axis_index("s")
    sl = pl.ds(s * ROWS_PER_TILE, ROWS_PER_TILE)
    pltpu.sync_copy(z2, ys.at[sl])
    plsc.subcore_barrier()

    def run(xs_hbm, src_hbm, dst_hbm):
        def group(gi, carry):
            at_stage = gi % GSTAGE == 0

            @pl.when(at_stage)
            def _():
                @pl.when(gi > 0)
                def _():
                    for b in range(NBUF):
                        pltpu.make_async_copy(
                            bufs[b], ys.at[dst_v.at[b]], ssems[b]).wait()
                st = gi // GSTAGE
                pltpu.sync_copy(src_hbm.at[s * STAGES + st], src_v)
                pltpu.sync_copy(dst_hbm.at[s * STAGES + st], dst_v)

            jj = (gi % GSTAGE) * NBUF
            for b in range(NBUF):
                @pl.when(jnp.logical_not(at_stage))
                def _():
                    pltpu.make_async_copy(
                        bufs[b], ys.at[dst_v.at[jj + b]], ssems[b]).wait()
                pltpu.async_copy(xs_hbm.at[src_v.at[jj + b]], bufs[b], gsems[b])
            for b in range(NBUF):
                pltpu.make_async_copy(
                    xs_hbm.at[src_v.at[jj + b]], bufs[b], gsems[b]).wait()
                pltpu.async_copy(bufs[b], ys.at[dst_v.at[jj + b]], ssems[b],
                                 add=True)
            return carry
        lax.fori_loop(0, GROUPS, group, 0)
        for b in range(NBUF):
            pltpu.make_async_copy(
                bufs[b], ys.at[dst_v.at[b]], ssems[b]).wait()

    @pl.when(c == 0)
    def _():
        run(xs0, src0, dst0)

    @pl.when(c == 1)
    def _():
        run(xs1, src1, dst1)

    plsc.subcore_barrier()

    @pl.when(c == 0)
    def _():
        pltpu.sync_copy(ys.at[sl], y0.at[sl])

    @pl.when(c == 1)
    def _():
        pltpu.sync_copy(ys.at[sl], y1.at[sl])


def _scale_body(x_ref, d0_ref, d1_ref, xs0_ref, xs1_ref):
    x = x_ref[...]
    n0 = lax.rsqrt(jnp.maximum(d0_ref[...], 1.0))
    n1 = lax.rsqrt(jnp.maximum(d1_ref[...], 1.0))
    xs0_ref[...] = x * n0
    xs1_ref[...] = x * n1


_RB = 2560
_NB = NPAD // _RB


def _dense_body(y0_ref, y1_ref, d0_ref, d1_ref, Wg0_ref, Wg1_ref, Ws1_ref,
                bs1_ref, w2_ref, Wp_ref, bg0_ref, bg1_ref, bp_ref, o_ref,
                acc, M0s, M1s, k0s, k1s, G0s, G1s, g0s, g1s):
    p = pl.program_id(0)
    i = pl.program_id(1)

    @pl.when((p == 0) & (i == 0))
    def _():
        M0s[...] = jnp.dot(Wg0_ref[...], Ws1_ref[...],
                           preferred_element_type=f32)
        M1s[...] = jnp.dot(Wg1_ref[...], Ws1_ref[...],
                           preferred_element_type=f32)
        k0s[...] = jnp.dot(bg0_ref[...], Ws1_ref[...],
                           preferred_element_type=f32) + bs1_ref[...]
        k1s[...] = jnp.dot(bg1_ref[...], Ws1_ref[...],
                           preferred_element_type=f32) + bs1_ref[...]
        G0s[...] = jnp.dot(Wg0_ref[...], Wp_ref[...],
                           preferred_element_type=f32)
        G1s[...] = jnp.dot(Wg1_ref[...], Wp_ref[...],
                           preferred_element_type=f32)
        g0s[...] = jnp.dot(bg0_ref[...], Wp_ref[...],
                           preferred_element_type=f32)
        g1s[...] = jnp.dot(bg1_ref[...], Wp_ref[...],
                           preferred_element_type=f32)
        acc[...] = jnp.zeros((8, 128), f32)

    yb0 = y0_ref[...] * lax.rsqrt(jnp.maximum(d0_ref[...], 1.0))
    yb1 = y1_ref[...] * lax.rsqrt(jnp.maximum(d1_ref[...], 1.0))
    row = lax.broadcasted_iota(i32, (8, 128), 0)
    lane = lax.broadcasted_iota(i32, (8, 128), 1)

    @pl.when(p == 0)
    def _():
        rows = lax.broadcasted_iota(i32, (_RB, 1), 0) + i * _RB
        mask = rows < N

        def part(yb, M_ref, k_ref):
            a = jnp.tanh(jnp.dot(yb, M_ref[...], preferred_element_type=f32)
                         + k_ref[...])
            t = jnp.sum(a * w2_ref[...], axis=1, keepdims=True)
            return jnp.sum(jnp.where(mask, t, 0.0))

        s0 = part(yb0, M0s, k0s)
        s1 = part(yb1, M1s, k1s)
        acc[...] += (jnp.where((row == 0) & (lane == 0), s0, 0.0)
                     + jnp.where((row == 1) & (lane == 0), s1, 0.0))

    @pl.when(p == 1)
    def _():
        srow = acc[...]
        w0 = jnp.sum(jnp.where((row == 0) & (lane == 0), srow, 0.0)) / N
        w1 = jnp.sum(jnp.where((row == 1) & (lane == 0), srow, 0.0)) / N
        m = jnp.maximum(w0, w1)
        e0 = jnp.exp(w0 - m)
        e1 = jnp.exp(w1 - m)
        b0 = e0 / (e0 + e1)
        b1 = e1 / (e0 + e1)
        p0 = jnp.dot(yb0, G0s[...], preferred_element_type=f32) + g0s[...]
        p1 = jnp.dot(yb1, G1s[...], preferred_element_type=f32) + g1s[...]
        o_ref[...] = b0 * p0 + b1 * p1 + bp_ref[...]


def _pad_edges(idx):
    pad = jnp.full((EPAD - E,), N, dtype=i32)
    return jnp.concatenate([idx, pad]).reshape(NTILES * STAGES, STAGE, CHUNK)


def kernel(x, edge_index0, edge_index1, Wg0, bg0, Wg1, bg1,
           Ws1, bs1, Ws2, Wp, bp):
    src0 = _pad_edges(edge_index0[0])
    dst0 = _pad_edges(edge_index0[1])
    src1 = _pad_edges(edge_index1[0])
    dst1 = _pad_edges(edge_index1[1])
    x_pad = jnp.pad(x, ((0, NPAD - N), (0, 0)))
    z1 = jnp.zeros((ROWS_PER_TILE,), f32)
    z2 = jnp.zeros((ROWS_PER_TILE, IN), f32)

    ds0, di0, ds1, di1 = _hist_kernel(src0, dst0, src1, dst1, z1)

    xs0, xs1 = pl.pallas_call(
        _scale_body,
        grid=(_NB,),
        in_specs=[
            pl.BlockSpec((_RB, IN), lambda i: (i, 0)),
            pl.BlockSpec((_RB, 1), lambda i: (i, 0)),
            pl.BlockSpec((_RB, 1), lambda i: (i, 0)),
        ],
        out_specs=[
            pl.BlockSpec((_RB, IN), lambda i: (i, 0)),
            pl.BlockSpec((_RB, IN), lambda i: (i, 0)),
        ],
        out_shape=[jax.ShapeDtypeStruct((NPAD, IN), f32)] * 2,
    )(x_pad, ds0.reshape(NPAD, 1), ds1.reshape(NPAD, 1))

    y0, y1 = _scatter_kernel(xs0, xs1, src0, dst0, src1, dst1, z2)

    di0c = di0.reshape(NPAD, 1)
    di1c = di1.reshape(NPAD, 1)

    DH = Wg0.shape[1]
    blk = lambda p, i: (i, 0)
    full = lambda p, i: (0, 0)
    outp = pl.pallas_call(
        _dense_body,
        grid=(2, _NB),
        in_specs=[
            pl.BlockSpec((_RB, IN), blk),
            pl.BlockSpec((_RB, IN), blk),
            pl.BlockSpec((_RB, 1), blk),
            pl.BlockSpec((_RB, 1), blk),
            pl.BlockSpec((IN, DH), full),
            pl.BlockSpec((IN, DH), full),
            pl.BlockSpec((DH, SEM_HID), full),
            pl.BlockSpec((1, SEM_HID), full),
            pl.BlockSpec((1, SEM_HID), full),
            pl.BlockSpec((DH, OUT), full),
            pl.BlockSpec((1, DH), full),
            pl.BlockSpec((1, DH), full),
            pl.BlockSpec((1, OUT), full),
        ],
        out_specs=pl.BlockSpec((_RB, OUT), blk),
        out_shape=jax.ShapeDtypeStruct((NPAD, OUT), f32),
        scratch_shapes=[
            pltpu.VMEM((8, 128), f32),
            pltpu.VMEM((IN, SEM_HID), f32),
            pltpu.VMEM((IN, SEM_HID), f32),
            pltpu.VMEM((1, SEM_HID), f32),
            pltpu.VMEM((1, SEM_HID), f32),
            pltpu.VMEM((IN, OUT), f32),
            pltpu.VMEM((IN, OUT), f32),
            pltpu.VMEM((1, OUT), f32),
            pltpu.VMEM((1, OUT), f32),
        ],
    )(y0, y1, di0c, di1c, Wg0, Wg1, Ws1,
      bs1.reshape(1, SEM_HID), Ws2.reshape(1, SEM_HID), Wp,
      bg0.reshape(1, DH), bg1.reshape(1, DH), bp.reshape(1, OUT))

    return outp[:N]
